# baseline (speedup 1.0000x reference)
import functools

import jax
import jax.numpy as jnp
from jax import lax
from jax.experimental import pallas as pl
from jax.experimental.pallas import tpu as pltpu
from jax.experimental.pallas import tpu_sc as plsc

B, S, D, E, TOPK = 1, 2048, 1024, 8, 2
DFF = 4 * D
EP = 128
T = B * S
TK = T * TOPK
TT = 256
NT = TK // TT
NWI = NT + E - 1
ND = 4
DC = DFF // ND
TSH = 8

NC, NS = 2, 16
NW = NC * NS
RC = 32

_NEG = -1e30


def _router_body(lg_ref, i1_ref, i2_ref, w1_ref, w2_ref):
    logits = lg_ref[...]
    lane = lax.broadcasted_iota(jnp.int32, (T, EP), 1)
    m1 = jnp.max(logits, axis=1, keepdims=True)
    i1 = jnp.min(jnp.where(logits == m1, lane, EP), axis=1, keepdims=True)
    l2 = jnp.where(lane == i1, _NEG, logits)
    m2 = jnp.max(l2, axis=1, keepdims=True)
    i2 = jnp.min(jnp.where(l2 == m2, lane, EP), axis=1, keepdims=True)
    ex = jnp.exp(m2 - m1)
    s1 = 1.0 / (1.0 + ex)
    i1_ref[...] = i1
    i2_ref[...] = i2
    w1_ref[...] = s1
    w2_ref[...] = 1.0 - s1


def _rank_body(eid_ref, dest_ref, wi_ref):
    eid = eid_ref[...]
    a = lax.broadcasted_iota(jnp.int32, (128, 128), 0)
    b = lax.broadcasted_iota(jnp.int32, (128, 128), 1)
    LT = (a < b).astype(jnp.float32)
    ar = lax.broadcasted_iota(jnp.int32, (32, 32), 0)
    br = lax.broadcasted_iota(jnp.int32, (32, 32), 1)
    LS = (br < ar).astype(jnp.float32)
    ii = lax.broadcasted_iota(jnp.int32, (1, 32), 1)

    destf = jnp.zeros((32, 128), jnp.float32)
    wi_t = jnp.zeros((1, 32), jnp.int32)
    wi_e = jnp.zeros((1, 32), jnp.int32)
    wi_lo = jnp.zeros((1, 32), jnp.int32)
    wi_hi = jnp.zeros((1, 32), jnp.int32)
    off_f = jnp.float32(0.0)
    off_i = jnp.int32(0)
    cum = jnp.int32(0)
    emax = jnp.int32(0)
    for e in range(E):
        mf = (eid == e).astype(jnp.float32)
        intra = jnp.dot(mf, LT, preferred_element_type=jnp.float32)
        rows = jnp.sum(mf, axis=1, keepdims=True)
        rowoff = jnp.dot(LS, rows, preferred_element_type=jnp.float32)
        destf = destf + mf * (off_f + intra + rowoff)
        cnt_f = jnp.sum(mf)
        cnt = cnt_f.astype(jnp.int32)
        ft = off_i >> TSH
        lt_ = (off_i + cnt - 1) >> TSH
        nt = jnp.where(cnt > 0, lt_ - ft + 1, 0)
        sel = (ii >= cum) & (ii < cum + nt)
        tile_e = ft + (ii - cum)
        wi_t = jnp.where(sel, tile_e, wi_t)
        wi_e = jnp.where(sel, e, wi_e)
        wi_lo = jnp.where(sel, jnp.maximum(off_i, tile_e << TSH), wi_lo)
        wi_hi = jnp.where(sel, jnp.minimum(off_i + cnt, (tile_e + 1) << TSH), wi_hi)
        emax = jnp.where(cnt > 0, e, emax)
        cum = cum + nt
        off_f = off_f + cnt_f
        off_i = off_i + cnt
    pad = ii >= cum
    wi_t = jnp.where(pad, NT - 1, wi_t)
    wi_e = jnp.where(pad, emax, wi_e)
    wi_lo = jnp.where(pad, 0, wi_lo)
    wi_hi = jnp.where(pad, 0, wi_hi)
    dest_ref[...] = destf.astype(jnp.int32)
    wi_ref[...] = jnp.concatenate([wi_t, wi_e, wi_lo, wi_hi], axis=0)


def _disp_body(x_hbm, dest_hbm, wrep_hbm, xg_hbm, ws_hbm,
               rows_v, idx_v, wv_v, sem, sem2):
    wid = lax.axis_index("s") * NC + lax.axis_index("c")
    tok0 = (wid % (T // 128)) * 128
    base0 = wid * 128
    for c in range(128 // RC):
        base = base0 + c * RC
        tok = tok0 + c * RC
        pltpu.sync_copy(x_hbm.at[pl.ds(tok, RC)], rows_v)
        pltpu.sync_copy(dest_hbm.at[pl.ds(base, RC)], idx_v)
        pltpu.sync_copy(wrep_hbm.at[pl.ds(base, RC)], wv_v)
        cp1 = pltpu.async_copy(rows_v, xg_hbm.at[idx_v], sem)
        cp2 = pltpu.async_copy(wv_v, ws_hbm.at[idx_v], sem2)
        cp1.wait()
        cp2.wait()


def _gmm_body(wt_ref, we_ref, lo_ref, hi_ref,
              xg_hbm, w1_ref, b1_ref, w2_ref, b2_ref, ws_ref, out_ref,
              acc_ref, xgv_ref, xsem):
    d = pl.program_id(0)
    i = pl.program_id(1)
    lo = lo_ref[i]
    hi = hi_ref[i]
    tile = wt_ref[i]

    @pl.when(jnp.logical_and(d == 0, i == 0))
    def _():
        cp = pltpu.make_async_copy(xg_hbm, xgv_ref, xsem)
        cp.start()
        cp.wait()

    @pl.when(hi > lo)
    def _():
        row = tile * TT + lax.broadcasted_iota(jnp.int32, (TT, 1), 0)
        mask = (row >= lo) & (row < hi)
        base = pl.multiple_of(tile * TT, TT)
        h = jnp.dot(xgv_ref[pl.ds(base, TT), :], w1_ref[0],
                    preferred_element_type=jnp.float32)
        h = h + b1_ref[0]
        h = h * (1.0 / (1.0 + jnp.exp(-h)))
        contrib = jnp.dot(h, w2_ref[0], preferred_element_type=jnp.float32)
        aslice = acc_ref[pl.ds(base, TT), :]

        @pl.when(d < ND - 1)
        def _():
            acc_ref[pl.ds(base, TT), :] = jnp.where(
                mask, jnp.where(d == 0, contrib, aslice + contrib), aslice)

        @pl.when(d == ND - 1)
        def _():
            wcol = ws_ref[...][:, :1]
            out_ref[...] = jnp.where(
                mask, (aslice + contrib + b2_ref[0]) * wcol, out_ref[...])


def _comb_body(y_hbm, dest_hbm, out_hbm, r0_v, r1_v, o_v, i0_v, i1_v, s0, s1):
    wid = lax.axis_index("s") * NC + lax.axis_index("c")
    tok0 = wid * (T // NW)
    for c in range((T // NW) // RC):
        tok = tok0 + c * RC
        pltpu.sync_copy(dest_hbm.at[pl.ds(tok, RC)], i0_v)
        pltpu.sync_copy(dest_hbm.at[pl.ds(T + tok, RC)], i1_v)
        cp0 = pltpu.async_copy(y_hbm.at[i0_v], r0_v, s0)
        cp1 = pltpu.async_copy(y_hbm.at[i1_v], r1_v, s1)
        cp0.wait()
        cp1.wait()

        def body(j, carry):
            r = j >> 6
            cc = pl.multiple_of((j & 63) << 4, 16)
            o_v[r, pl.ds(cc, 16)] = r0_v[r, pl.ds(cc, 16)] + r1_v[r, pl.ds(cc, 16)]
            return carry

        lax.fori_loop(0, RC * (D // 16), body, 0)
        pltpu.sync_copy(o_v, out_hbm.at[pl.ds(tok, RC)])


@jax.jit
def kernel(x, Wg, bg, Wng, bng, W1, b1, W2, b2):
    xf = x.reshape(T, D)
    gate = jnp.einsum('bsd,de->bse', x, Wg) + bg
    noise = jax.random.normal(jax.random.key(42), gate.shape, dtype=gate.dtype)
    logits = gate + noise + jnp.einsum('bsd,de->bse', x, Wng) + bng
    lgp = jnp.full((T, EP), _NEG, jnp.float32).at[:, :E].set(logits.reshape(T, E))

    i1, i2, w1_, w2_ = pl.pallas_call(
        _router_body,
        out_shape=[
            jax.ShapeDtypeStruct((T, 1), jnp.int32),
            jax.ShapeDtypeStruct((T, 1), jnp.int32),
            jax.ShapeDtypeStruct((T, 1), jnp.float32),
            jax.ShapeDtypeStruct((T, 1), jnp.float32),
        ],
    )(lgp)

    eidp = jnp.concatenate([i1[:, 0], i2[:, 0]]).reshape(32, 128)
    wflat = jnp.concatenate([w1_[:, 0], w2_[:, 0]])
    wrep = jnp.broadcast_to(wflat[:, None], (TK, 128))

    dest32, wi = pl.pallas_call(
        _rank_body,
        out_shape=[
            jax.ShapeDtypeStruct((32, 128), jnp.int32),
            jax.ShapeDtypeStruct((4, 32), jnp.int32),
        ],
    )(eidp)
    dest = dest32.reshape(TK)
    wi_t, wi_e, wi_lo, wi_hi = wi[0], wi[1], wi[2], wi[3]

    xg, wsrt = pl.kernel(
        _disp_body,
        out_type=[
            jax.ShapeDtypeStruct((TK, D), jnp.float32),
            jax.ShapeDtypeStruct((TK, 128), jnp.float32),
        ],
        mesh=plsc.VectorSubcoreMesh(core_axis_name="c", subcore_axis_name="s", num_cores=NC, num_subcores=NS),
        scratch_types=[
            pltpu.VMEM((RC, D), jnp.float32),
            pltpu.VMEM((RC,), jnp.int32),
            pltpu.VMEM((RC, 128), jnp.float32),
            pltpu.SemaphoreType.DMA,
            pltpu.SemaphoreType.DMA,
        ],
    )(xf, dest, wrep)

    ysort = pl.pallas_call(
        _gmm_body,
        grid_spec=pltpu.PrefetchScalarGridSpec(
            num_scalar_prefetch=4,
            grid=(ND, NWI),
            in_specs=[
                pl.BlockSpec(memory_space=pltpu.MemorySpace.HBM),
                pl.BlockSpec((1, D, DC), lambda d, i, wt, we, lo, hi:
                             (we[i], 0, d)),
                pl.BlockSpec((1, 1, DC), lambda d, i, wt, we, lo, hi:
                             (we[i], 0, d)),
                pl.BlockSpec((1, DC, D), lambda d, i, wt, we, lo, hi:
                             (we[i], d, 0)),
                pl.BlockSpec((1, 1, D), lambda d, i, wt, we, lo, hi: (we[i], 0, 0)),
                pl.BlockSpec((TT, 128), lambda d, i, wt, we, lo, hi: (wt[i], 0)),
            ],
            out_specs=pl.BlockSpec(
                (TT, D),
                lambda d, i, wt, we, lo, hi: (jnp.where(d == ND - 1, wt[i], 0), 0)),
            scratch_shapes=[pltpu.VMEM((TK, D), jnp.float32),
                            pltpu.VMEM((TK, D), jnp.float32),
                            pltpu.SemaphoreType.DMA],
        ),
        out_shape=jax.ShapeDtypeStruct((TK, D), jnp.float32),
        compiler_params=pltpu.CompilerParams(
            dimension_semantics=("arbitrary", "arbitrary"),
        ),
    )(wi_t, wi_e, wi_lo, wi_hi, xg, W1, b1.reshape(E, 1, DFF), W2,
      b2.reshape(E, 1, D), wsrt)

    out = pl.kernel(
        _comb_body,
        out_type=jax.ShapeDtypeStruct((T, D), jnp.float32),
        mesh=plsc.VectorSubcoreMesh(core_axis_name="c", subcore_axis_name="s", num_cores=NC, num_subcores=NS),
        scratch_types=[
            pltpu.VMEM((RC, D), jnp.float32),
            pltpu.VMEM((RC, D), jnp.float32),
            pltpu.VMEM((RC, D), jnp.float32),
            pltpu.VMEM((RC,), jnp.int32),
            pltpu.VMEM((RC,), jnp.int32),
            pltpu.SemaphoreType.DMA,
            pltpu.SemaphoreType.DMA,
        ],
    )(ysort, dest)

    return out.reshape(B, S, D)


# speedup vs baseline: 1.1361x; 1.1361x over previous
import functools

import jax
import jax.numpy as jnp
from jax import lax
from jax.experimental import pallas as pl
from jax.experimental.pallas import tpu as pltpu
from jax.experimental.pallas import tpu_sc as plsc

B, S, D, E, TOPK = 1, 2048, 1024, 8, 2
DFF = 4 * D
EP = 128
T = B * S
TK = T * TOPK
TT = 256
NT = TK // TT
NWI = NT + E - 1
ND = 2
DC = DFF // ND
TSH = 8

NC, NS = 2, 16
NW = NC * NS
RC = 32

_NEG = -1e30


def _router_body(lg_ref, i1_ref, i2_ref, w1_ref, w2_ref):
    logits = lg_ref[...]
    lane = lax.broadcasted_iota(jnp.int32, (T, EP), 1)
    m1 = jnp.max(logits, axis=1, keepdims=True)
    i1 = jnp.min(jnp.where(logits == m1, lane, EP), axis=1, keepdims=True)
    l2 = jnp.where(lane == i1, _NEG, logits)
    m2 = jnp.max(l2, axis=1, keepdims=True)
    i2 = jnp.min(jnp.where(l2 == m2, lane, EP), axis=1, keepdims=True)
    ex = jnp.exp(m2 - m1)
    s1 = 1.0 / (1.0 + ex)
    i1_ref[...] = i1
    i2_ref[...] = i2
    w1_ref[...] = s1
    w2_ref[...] = 1.0 - s1


def _rank_body(eid_ref, dest_ref, wi_ref):
    eid = eid_ref[...]
    a = lax.broadcasted_iota(jnp.int32, (128, 128), 0)
    b = lax.broadcasted_iota(jnp.int32, (128, 128), 1)
    LT = (a < b).astype(jnp.float32)
    ar = lax.broadcasted_iota(jnp.int32, (32, 32), 0)
    br = lax.broadcasted_iota(jnp.int32, (32, 32), 1)
    LS = (br < ar).astype(jnp.float32)
    ii = lax.broadcasted_iota(jnp.int32, (1, 32), 1)

    destf = jnp.zeros((32, 128), jnp.float32)
    wi_t = jnp.zeros((1, 32), jnp.int32)
    wi_e = jnp.zeros((1, 32), jnp.int32)
    wi_lo = jnp.zeros((1, 32), jnp.int32)
    wi_hi = jnp.zeros((1, 32), jnp.int32)
    off_f = jnp.float32(0.0)
    off_i = jnp.int32(0)
    cum = jnp.int32(0)
    emax = jnp.int32(0)
    for e in range(E):
        mf = (eid == e).astype(jnp.float32)
        intra = jnp.dot(mf, LT, preferred_element_type=jnp.float32)
        rows = jnp.sum(mf, axis=1, keepdims=True)
        rowoff = jnp.dot(LS, rows, preferred_element_type=jnp.float32)
        destf = destf + mf * (off_f + intra + rowoff)
        cnt_f = jnp.sum(mf)
        cnt = cnt_f.astype(jnp.int32)
        ft = off_i >> TSH
        lt_ = (off_i + cnt - 1) >> TSH
        nt = jnp.where(cnt > 0, lt_ - ft + 1, 0)
        sel = (ii >= cum) & (ii < cum + nt)
        tile_e = ft + (ii - cum)
        wi_t = jnp.where(sel, tile_e, wi_t)
        wi_e = jnp.where(sel, e, wi_e)
        wi_lo = jnp.where(sel, jnp.maximum(off_i, tile_e << TSH), wi_lo)
        wi_hi = jnp.where(sel, jnp.minimum(off_i + cnt, (tile_e + 1) << TSH), wi_hi)
        emax = jnp.where(cnt > 0, e, emax)
        cum = cum + nt
        off_f = off_f + cnt_f
        off_i = off_i + cnt
    pad = ii >= cum
    wi_t = jnp.where(pad, NT - 1, wi_t)
    wi_e = jnp.where(pad, emax, wi_e)
    wi_lo = jnp.where(pad, 0, wi_lo)
    wi_hi = jnp.where(pad, 0, wi_hi)
    dest_ref[...] = destf.astype(jnp.int32)
    wi_ref[...] = jnp.concatenate([wi_t, wi_e, wi_lo, wi_hi], axis=0)


def _disp_body(x_hbm, dest_hbm, wrep_hbm, xg_hbm, ws_hbm,
               rows_v, idx_v, wv_v, sem, sem2):
    wid = lax.axis_index("s") * NC + lax.axis_index("c")
    tok0 = (wid % (T // 128)) * 128
    base0 = wid * 128
    for c in range(128 // RC):
        base = base0 + c * RC
        tok = tok0 + c * RC
        pltpu.sync_copy(x_hbm.at[pl.ds(tok, RC)], rows_v)
        pltpu.sync_copy(dest_hbm.at[pl.ds(base, RC)], idx_v)
        pltpu.sync_copy(wrep_hbm.at[pl.ds(base, RC)], wv_v)
        cp1 = pltpu.async_copy(rows_v, xg_hbm.at[idx_v], sem)
        cp2 = pltpu.async_copy(wv_v, ws_hbm.at[idx_v], sem2)
        cp1.wait()
        cp2.wait()


def _gmm_body(wt_ref, we_ref, lo_ref, hi_ref,
              xg_ref, w1_ref, b1_ref, w2_ref, b2_ref, ws_ref, out_ref,
              acc_ref):
    d = pl.program_id(0)
    i = pl.program_id(1)
    lo = lo_ref[i]
    hi = hi_ref[i]
    tile = wt_ref[i]

    @pl.when(hi > lo)
    def _():
        row = tile * TT + lax.broadcasted_iota(jnp.int32, (TT, 1), 0)
        mask = (row >= lo) & (row < hi)
        base = pl.multiple_of(tile * TT, TT)
        h = jnp.dot(xg_ref[...], w1_ref[0],
                    preferred_element_type=jnp.float32)
        h = h + b1_ref[0]
        h = h * (1.0 / (1.0 + jnp.exp(-h)))
        contrib = jnp.dot(h, w2_ref[0], preferred_element_type=jnp.float32)
        aslice = acc_ref[pl.ds(base, TT), :]

        @pl.when(d < ND - 1)
        def _():
            acc_ref[pl.ds(base, TT), :] = jnp.where(
                mask, jnp.where(d == 0, contrib, aslice + contrib), aslice)

        @pl.when(d == ND - 1)
        def _():
            wcol = ws_ref[...][:, :1]
            out_ref[...] = jnp.where(
                mask, (aslice + contrib + b2_ref[0]) * wcol, out_ref[...])


def _comb_body(y_hbm, dest_hbm, out_hbm, r0_v, r1_v, o_v, i0_v, i1_v, s0, s1):
    wid = lax.axis_index("s") * NC + lax.axis_index("c")
    tok0 = wid * (T // NW)
    for c in range((T // NW) // RC):
        tok = tok0 + c * RC
        pltpu.sync_copy(dest_hbm.at[pl.ds(tok, RC)], i0_v)
        pltpu.sync_copy(dest_hbm.at[pl.ds(T + tok, RC)], i1_v)
        cp0 = pltpu.async_copy(y_hbm.at[i0_v], r0_v, s0)
        cp1 = pltpu.async_copy(y_hbm.at[i1_v], r1_v, s1)
        cp0.wait()
        cp1.wait()

        def body(j, carry):
            r = j >> 6
            cc = pl.multiple_of((j & 63) << 4, 16)
            o_v[r, pl.ds(cc, 16)] = r0_v[r, pl.ds(cc, 16)] + r1_v[r, pl.ds(cc, 16)]
            return carry

        lax.fori_loop(0, RC * (D // 16), body, 0)
        pltpu.sync_copy(o_v, out_hbm.at[pl.ds(tok, RC)])


@jax.jit
def kernel(x, Wg, bg, Wng, bng, W1, b1, W2, b2):
    xf = x.reshape(T, D)
    gate = jnp.einsum('bsd,de->bse', x, Wg) + bg
    noise = jax.random.normal(jax.random.key(42), gate.shape, dtype=gate.dtype)
    logits = gate + noise + jnp.einsum('bsd,de->bse', x, Wng) + bng
    lgp = jnp.full((T, EP), _NEG, jnp.float32).at[:, :E].set(logits.reshape(T, E))

    i1, i2, w1_, w2_ = pl.pallas_call(
        _router_body,
        out_shape=[
            jax.ShapeDtypeStruct((T, 1), jnp.int32),
            jax.ShapeDtypeStruct((T, 1), jnp.int32),
            jax.ShapeDtypeStruct((T, 1), jnp.float32),
            jax.ShapeDtypeStruct((T, 1), jnp.float32),
        ],
    )(lgp)

    eidp = jnp.concatenate([i1[:, 0], i2[:, 0]]).reshape(32, 128)
    wflat = jnp.concatenate([w1_[:, 0], w2_[:, 0]])
    wrep = jnp.broadcast_to(wflat[:, None], (TK, 128))

    dest32, wi = pl.pallas_call(
        _rank_body,
        out_shape=[
            jax.ShapeDtypeStruct((32, 128), jnp.int32),
            jax.ShapeDtypeStruct((4, 32), jnp.int32),
        ],
    )(eidp)
    dest = dest32.reshape(TK)
    wi_t, wi_e, wi_lo, wi_hi = wi[0], wi[1], wi[2], wi[3]

    xg, wsrt = pl.kernel(
        _disp_body,
        out_type=[
            jax.ShapeDtypeStruct((TK, D), jnp.float32),
            jax.ShapeDtypeStruct((TK, 128), jnp.float32),
        ],
        mesh=plsc.VectorSubcoreMesh(core_axis_name="c", subcore_axis_name="s", num_cores=NC, num_subcores=NS),
        scratch_types=[
            pltpu.VMEM((RC, D), jnp.float32),
            pltpu.VMEM((RC,), jnp.int32),
            pltpu.VMEM((RC, 128), jnp.float32),
            pltpu.SemaphoreType.DMA,
            pltpu.SemaphoreType.DMA,
        ],
    )(xf, dest, wrep)

    ysort = pl.pallas_call(
        _gmm_body,
        grid_spec=pltpu.PrefetchScalarGridSpec(
            num_scalar_prefetch=4,
            grid=(ND, NWI),
            in_specs=[
                pl.BlockSpec((TT, D), lambda d, i, wt, we, lo, hi: (wt[i], 0)),
                pl.BlockSpec((1, D, DC), lambda d, i, wt, we, lo, hi:
                             (we[i], 0, d)),
                pl.BlockSpec((1, 1, DC), lambda d, i, wt, we, lo, hi:
                             (we[i], 0, d)),
                pl.BlockSpec((1, DC, D), lambda d, i, wt, we, lo, hi:
                             (we[i], d, 0)),
                pl.BlockSpec((1, 1, D), lambda d, i, wt, we, lo, hi: (we[i], 0, 0)),
                pl.BlockSpec((TT, 128), lambda d, i, wt, we, lo, hi: (wt[i], 0)),
            ],
            out_specs=pl.BlockSpec(
                (TT, D),
                lambda d, i, wt, we, lo, hi: (jnp.where(d == ND - 1, wt[i], 0), 0)),
            scratch_shapes=[pltpu.VMEM((TK, D), jnp.float32)],
        ),
        out_shape=jax.ShapeDtypeStruct((TK, D), jnp.float32),
        compiler_params=pltpu.CompilerParams(
            dimension_semantics=("arbitrary", "arbitrary"),
        ),
    )(wi_t, wi_e, wi_lo, wi_hi, xg, W1, b1.reshape(E, 1, DFF), W2,
      b2.reshape(E, 1, D), wsrt)

    out = pl.kernel(
        _comb_body,
        out_type=jax.ShapeDtypeStruct((T, D), jnp.float32),
        mesh=plsc.VectorSubcoreMesh(core_axis_name="c", subcore_axis_name="s", num_cores=NC, num_subcores=NS),
        scratch_types=[
            pltpu.VMEM((RC, D), jnp.float32),
            pltpu.VMEM((RC, D), jnp.float32),
            pltpu.VMEM((RC, D), jnp.float32),
            pltpu.VMEM((RC,), jnp.int32),
            pltpu.VMEM((RC,), jnp.int32),
            pltpu.SemaphoreType.DMA,
            pltpu.SemaphoreType.DMA,
        ],
    )(ysort, dest)

    return out.reshape(B, S, D)


# speedup vs baseline: 1.1536x; 1.0154x over previous
import functools

import jax
import jax.numpy as jnp
from jax import lax
from jax.experimental import pallas as pl
from jax.experimental.pallas import tpu as pltpu
from jax.experimental.pallas import tpu_sc as plsc

B, S, D, E, TOPK = 1, 2048, 1024, 8, 2
DFF = 4 * D
EP = 128
T = B * S
TK = T * TOPK
TT = 256
NT = TK // TT
NWI = NT + E - 1
ND = 2
DC = DFF // ND
TSH = 8

NC, NS = 2, 16
NW = NC * NS
RC = 32

_NEG = -1e30


def _router_body(lg_ref, i1_ref, i2_ref, w1_ref, w2_ref):
    logits = lg_ref[...]
    lane = lax.broadcasted_iota(jnp.int32, (T, EP), 1)
    m1 = jnp.max(logits, axis=1, keepdims=True)
    i1 = jnp.min(jnp.where(logits == m1, lane, EP), axis=1, keepdims=True)
    l2 = jnp.where(lane == i1, _NEG, logits)
    m2 = jnp.max(l2, axis=1, keepdims=True)
    i2 = jnp.min(jnp.where(l2 == m2, lane, EP), axis=1, keepdims=True)
    ex = jnp.exp(m2 - m1)
    s1 = 1.0 / (1.0 + ex)
    i1_ref[...] = i1
    i2_ref[...] = i2
    w1_ref[...] = s1
    w2_ref[...] = 1.0 - s1


def _rank_body(eid_ref, dest_ref, wi_ref):
    eid = eid_ref[...]
    a = lax.broadcasted_iota(jnp.int32, (128, 128), 0)
    b = lax.broadcasted_iota(jnp.int32, (128, 128), 1)
    LT = (a < b).astype(jnp.float32)
    ar = lax.broadcasted_iota(jnp.int32, (32, 32), 0)
    br = lax.broadcasted_iota(jnp.int32, (32, 32), 1)
    LS = (br < ar).astype(jnp.float32)
    ii = lax.broadcasted_iota(jnp.int32, (1, 32), 1)

    destf = jnp.zeros((32, 128), jnp.float32)
    wi_t = jnp.zeros((1, 32), jnp.int32)
    wi_e = jnp.zeros((1, 32), jnp.int32)
    wi_lo = jnp.zeros((1, 32), jnp.int32)
    wi_hi = jnp.zeros((1, 32), jnp.int32)
    off_f = jnp.float32(0.0)
    off_i = jnp.int32(0)
    cum = jnp.int32(0)
    emax = jnp.int32(0)
    for e in range(E):
        mf = (eid == e).astype(jnp.float32)
        intra = jnp.dot(mf, LT, preferred_element_type=jnp.float32)
        rows = jnp.sum(mf, axis=1, keepdims=True)
        rowoff = jnp.dot(LS, rows, preferred_element_type=jnp.float32)
        destf = destf + mf * (off_f + intra + rowoff)
        cnt_f = jnp.sum(mf)
        cnt = cnt_f.astype(jnp.int32)
        ft = off_i >> TSH
        lt_ = (off_i + cnt - 1) >> TSH
        nt = jnp.where(cnt > 0, lt_ - ft + 1, 0)
        sel = (ii >= cum) & (ii < cum + nt)
        tile_e = ft + (ii - cum)
        wi_t = jnp.where(sel, tile_e, wi_t)
        wi_e = jnp.where(sel, e, wi_e)
        wi_lo = jnp.where(sel, jnp.maximum(off_i, tile_e << TSH), wi_lo)
        wi_hi = jnp.where(sel, jnp.minimum(off_i + cnt, (tile_e + 1) << TSH), wi_hi)
        emax = jnp.where(cnt > 0, e, emax)
        cum = cum + nt
        off_f = off_f + cnt_f
        off_i = off_i + cnt
    pad = ii >= cum
    wi_t = jnp.where(pad, NT - 1, wi_t)
    wi_e = jnp.where(pad, emax, wi_e)
    wi_lo = jnp.where(pad, 0, wi_lo)
    wi_hi = jnp.where(pad, 0, wi_hi)
    dest_ref[...] = destf.astype(jnp.int32)
    wi_ref[...] = jnp.concatenate([wi_t, wi_e, wi_lo, wi_hi], axis=0)


def _disp_body(x_hbm, dest_hbm, wrep_hbm, xg_hbm, ws_hbm,
               rows_a, rows_b, idx_a, idx_b, wv_a, wv_b, lsem, sem, sem2):
    wid = lax.axis_index("s") * NC + lax.axis_index("c")
    tok0 = (wid % (T // 128)) * 128
    base0 = wid * 128
    nch = 128 // RC
    bufs = [(rows_a, idx_a, wv_a), (rows_b, idx_b, wv_b)]

    def start_load(c, rows_v, idx_v, wv_v):
        base = base0 + c * RC
        tok = tok0 + c * RC
        return (pltpu.async_copy(x_hbm.at[pl.ds(tok, RC)], rows_v, lsem),
                pltpu.async_copy(dest_hbm.at[pl.ds(base, RC)], idx_v, lsem),
                pltpu.async_copy(wrep_hbm.at[pl.ds(base, RC)], wv_v, lsem))

    loads = start_load(0, *bufs[0])
    scat = [None, None]
    for c in range(nch):
        for cp in loads:
            cp.wait()
        rows_v, idx_v, wv_v = bufs[c % 2]
        scat[c % 2] = (pltpu.async_copy(rows_v, xg_hbm.at[idx_v], sem),
                       pltpu.async_copy(wv_v, ws_hbm.at[idx_v], sem2))
        if c + 1 < nch:
            b2 = (c + 1) % 2
            if scat[b2] is not None:
                for cp in scat[b2]:
                    cp.wait()
                scat[b2] = None
            loads = start_load(c + 1, *bufs[b2])
    for pair in scat:
        if pair is not None:
            for cp in pair:
                cp.wait()


def _gmm_body(wt_ref, we_ref, lo_ref, hi_ref,
              xg_ref, w1_ref, b1_ref, w2_ref, b2_ref, ws_ref, out_ref,
              acc_ref):
    d = pl.program_id(0)
    i = pl.program_id(1)
    lo = lo_ref[i]
    hi = hi_ref[i]
    tile = wt_ref[i]

    @pl.when(hi > lo)
    def _():
        row = tile * TT + lax.broadcasted_iota(jnp.int32, (TT, 1), 0)
        mask = (row >= lo) & (row < hi)
        base = pl.multiple_of(tile * TT, TT)
        h = jnp.dot(xg_ref[...], w1_ref[0],
                    preferred_element_type=jnp.float32)
        h = h + b1_ref[0]
        h = h * (1.0 / (1.0 + jnp.exp(-h)))
        contrib = jnp.dot(h, w2_ref[0], preferred_element_type=jnp.float32)
        aslice = acc_ref[pl.ds(base, TT), :]

        @pl.when(d < ND - 1)
        def _():
            acc_ref[pl.ds(base, TT), :] = jnp.where(
                mask, jnp.where(d == 0, contrib, aslice + contrib), aslice)

        @pl.when(d == ND - 1)
        def _():
            wcol = ws_ref[...][:, :1]
            out_ref[...] = jnp.where(
                mask, (aslice + contrib + b2_ref[0]) * wcol, out_ref[...])


RCC = 16


def _comb_body(y_hbm, dest_hbm, out_hbm,
               r0_a, r1_a, r0_b, r1_b, i0_a, i1_a, i0_b, i1_b, g0, g1):
    wid = lax.axis_index("s") * NC + lax.axis_index("c")
    tok0 = wid * (T // NW)
    nch = (T // NW) // RCC
    bufs = [(r0_a, r1_a, i0_a, i1_a), (r0_b, r1_b, i0_b, i1_b)]

    def start(c, r0_v, r1_v, i0_v, i1_v):
        tok = tok0 + c * RCC
        pltpu.sync_copy(dest_hbm.at[pl.ds(tok, RCC)], i0_v)
        pltpu.sync_copy(dest_hbm.at[pl.ds(T + tok, RCC)], i1_v)
        return (pltpu.async_copy(y_hbm.at[i0_v], r0_v, g0),
                pltpu.async_copy(y_hbm.at[i1_v], r1_v, g1))

    cps = start(0, *bufs[0])
    for c in range(nch):
        r0_v, r1_v, _, _ = bufs[c % 2]
        for cp in cps:
            cp.wait()
        if c + 1 < nch:
            cps = start(c + 1, *bufs[(c + 1) % 2])

        def body(j, carry):
            r = j >> 6
            cc = pl.multiple_of((j & 63) << 4, 16)
            r0_v[r, pl.ds(cc, 16)] = (r0_v[r, pl.ds(cc, 16)]
                                      + r1_v[r, pl.ds(cc, 16)])
            return carry

        lax.fori_loop(0, RCC * (D // 16), body, 0)
        pltpu.sync_copy(r0_v, out_hbm.at[pl.ds(tok0 + c * RCC, RCC)])


@jax.jit
def kernel(x, Wg, bg, Wng, bng, W1, b1, W2, b2):
    xf = x.reshape(T, D)
    gate = jnp.einsum('bsd,de->bse', x, Wg) + bg
    noise = jax.random.normal(jax.random.key(42), gate.shape, dtype=gate.dtype)
    logits = gate + noise + jnp.einsum('bsd,de->bse', x, Wng) + bng
    lgp = jnp.full((T, EP), _NEG, jnp.float32).at[:, :E].set(logits.reshape(T, E))

    i1, i2, w1_, w2_ = pl.pallas_call(
        _router_body,
        out_shape=[
            jax.ShapeDtypeStruct((T, 1), jnp.int32),
            jax.ShapeDtypeStruct((T, 1), jnp.int32),
            jax.ShapeDtypeStruct((T, 1), jnp.float32),
            jax.ShapeDtypeStruct((T, 1), jnp.float32),
        ],
    )(lgp)

    eidp = jnp.concatenate([i1[:, 0], i2[:, 0]]).reshape(32, 128)
    wflat = jnp.concatenate([w1_[:, 0], w2_[:, 0]])
    wrep = jnp.broadcast_to(wflat[:, None], (TK, 128))

    dest32, wi = pl.pallas_call(
        _rank_body,
        out_shape=[
            jax.ShapeDtypeStruct((32, 128), jnp.int32),
            jax.ShapeDtypeStruct((4, 32), jnp.int32),
        ],
    )(eidp)
    dest = dest32.reshape(TK)
    wi_t, wi_e, wi_lo, wi_hi = wi[0], wi[1], wi[2], wi[3]

    xg, wsrt = pl.kernel(
        _disp_body,
        out_type=[
            jax.ShapeDtypeStruct((TK, D), jnp.float32),
            jax.ShapeDtypeStruct((TK, 128), jnp.float32),
        ],
        mesh=plsc.VectorSubcoreMesh(core_axis_name="c", subcore_axis_name="s", num_cores=NC, num_subcores=NS),
        scratch_types=[
            pltpu.VMEM((RC, D), jnp.float32),
            pltpu.VMEM((RC, D), jnp.float32),
            pltpu.VMEM((RC,), jnp.int32),
            pltpu.VMEM((RC,), jnp.int32),
            pltpu.VMEM((RC, 128), jnp.float32),
            pltpu.VMEM((RC, 128), jnp.float32),
            pltpu.SemaphoreType.DMA,
            pltpu.SemaphoreType.DMA,
            pltpu.SemaphoreType.DMA,
        ],
    )(xf, dest, wrep)

    ysort = pl.pallas_call(
        _gmm_body,
        grid_spec=pltpu.PrefetchScalarGridSpec(
            num_scalar_prefetch=4,
            grid=(ND, NWI),
            in_specs=[
                pl.BlockSpec((TT, D), lambda d, i, wt, we, lo, hi: (wt[i], 0)),
                pl.BlockSpec((1, D, DC), lambda d, i, wt, we, lo, hi:
                             (we[i], 0, d)),
                pl.BlockSpec((1, 1, DC), lambda d, i, wt, we, lo, hi:
                             (we[i], 0, d)),
                pl.BlockSpec((1, DC, D), lambda d, i, wt, we, lo, hi:
                             (we[i], d, 0)),
                pl.BlockSpec((1, 1, D), lambda d, i, wt, we, lo, hi: (we[i], 0, 0)),
                pl.BlockSpec((TT, 128), lambda d, i, wt, we, lo, hi: (wt[i], 0)),
            ],
            out_specs=pl.BlockSpec(
                (TT, D),
                lambda d, i, wt, we, lo, hi: (jnp.where(d == ND - 1, wt[i], 0), 0)),
            scratch_shapes=[pltpu.VMEM((TK, D), jnp.float32)],
        ),
        out_shape=jax.ShapeDtypeStruct((TK, D), jnp.float32),
        compiler_params=pltpu.CompilerParams(
            dimension_semantics=("arbitrary", "arbitrary"),
        ),
    )(wi_t, wi_e, wi_lo, wi_hi, xg, W1, b1.reshape(E, 1, DFF), W2,
      b2.reshape(E, 1, D), wsrt)

    out = pl.kernel(
        _comb_body,
        out_type=jax.ShapeDtypeStruct((T, D), jnp.float32),
        mesh=plsc.VectorSubcoreMesh(core_axis_name="c", subcore_axis_name="s", num_cores=NC, num_subcores=NS),
        scratch_types=[
            pltpu.VMEM((RCC, D), jnp.float32),
            pltpu.VMEM((RCC, D), jnp.float32),
            pltpu.VMEM((RCC, D), jnp.float32),
            pltpu.VMEM((RCC, D), jnp.float32),
            pltpu.VMEM((RCC,), jnp.int32),
            pltpu.VMEM((RCC,), jnp.int32),
            pltpu.VMEM((RCC,), jnp.int32),
            pltpu.VMEM((RCC,), jnp.int32),
            pltpu.SemaphoreType.DMA,
            pltpu.SemaphoreType.DMA,
        ],
    )(ysort, dest)

    return out.reshape(B, S, D)


# speedup vs baseline: 1.2026x; 1.0425x over previous
import functools

import jax
import jax.numpy as jnp
from jax import lax
from jax.experimental import pallas as pl
from jax.experimental.pallas import tpu as pltpu
from jax.experimental.pallas import tpu_sc as plsc

B, S, D, E, TOPK = 1, 2048, 1024, 8, 2
DFF = 4 * D
EP = 128
T = B * S
TK = T * TOPK
TT = 256
NT = TK // TT
NWI = NT + E - 1
ND = 2
DC = DFF // ND
TSH = 8

NC, NS = 2, 16
NW = NC * NS
RC = 32

_NEG = -1e30


def _router_body(lg_ref, i1_ref, i2_ref, w1_ref, w2_ref):
    logits = lg_ref[...]
    lane = lax.broadcasted_iota(jnp.int32, (T, EP), 1)
    m1 = jnp.max(logits, axis=1, keepdims=True)
    i1 = jnp.min(jnp.where(logits == m1, lane, EP), axis=1, keepdims=True)
    l2 = jnp.where(lane == i1, _NEG, logits)
    m2 = jnp.max(l2, axis=1, keepdims=True)
    i2 = jnp.min(jnp.where(l2 == m2, lane, EP), axis=1, keepdims=True)
    ex = jnp.exp(m2 - m1)
    s1 = 1.0 / (1.0 + ex)
    i1_ref[...] = i1
    i2_ref[...] = i2
    w1_ref[...] = s1
    w2_ref[...] = 1.0 - s1


def _rank_body(eid_ref, dest_ref, wi_ref):
    eid = eid_ref[...]
    a = lax.broadcasted_iota(jnp.int32, (128, 128), 0)
    b = lax.broadcasted_iota(jnp.int32, (128, 128), 1)
    LT = (a < b).astype(jnp.float32)
    ar = lax.broadcasted_iota(jnp.int32, (32, 32), 0)
    br = lax.broadcasted_iota(jnp.int32, (32, 32), 1)
    LS = (br < ar).astype(jnp.float32)
    ii = lax.broadcasted_iota(jnp.int32, (1, 32), 1)

    destf = jnp.zeros((32, 128), jnp.float32)
    wi_t = jnp.zeros((1, 32), jnp.int32)
    wi_e = jnp.zeros((1, 32), jnp.int32)
    wi_lo = jnp.zeros((1, 32), jnp.int32)
    wi_hi = jnp.zeros((1, 32), jnp.int32)
    off_f = jnp.float32(0.0)
    off_i = jnp.int32(0)
    cum = jnp.int32(0)
    emax = jnp.int32(0)
    for e in range(E):
        mf = (eid == e).astype(jnp.float32)
        intra = jnp.dot(mf, LT, preferred_element_type=jnp.float32)
        rows = jnp.sum(mf, axis=1, keepdims=True)
        rowoff = jnp.dot(LS, rows, preferred_element_type=jnp.float32)
        destf = destf + mf * (off_f + intra + rowoff)
        cnt_f = jnp.sum(mf)
        cnt = cnt_f.astype(jnp.int32)
        ft = off_i >> TSH
        lt_ = (off_i + cnt - 1) >> TSH
        nt = jnp.where(cnt > 0, lt_ - ft + 1, 0)
        sel = (ii >= cum) & (ii < cum + nt)
        tile_e = ft + (ii - cum)
        wi_t = jnp.where(sel, tile_e, wi_t)
        wi_e = jnp.where(sel, e, wi_e)
        wi_lo = jnp.where(sel, jnp.maximum(off_i, tile_e << TSH), wi_lo)
        wi_hi = jnp.where(sel, jnp.minimum(off_i + cnt, (tile_e + 1) << TSH), wi_hi)
        emax = jnp.where(cnt > 0, e, emax)
        cum = cum + nt
        off_f = off_f + cnt_f
        off_i = off_i + cnt
    pad = ii >= cum
    wi_t = jnp.where(pad, NT - 1, wi_t)
    wi_e = jnp.where(pad, emax, wi_e)
    wi_lo = jnp.where(pad, 0, wi_lo)
    wi_hi = jnp.where(pad, 0, wi_hi)
    dest_ref[...] = destf.astype(jnp.int32)
    wi_ref[...] = jnp.concatenate([wi_t, wi_e, wi_lo, wi_hi], axis=0)


def _disp_body(x_hbm, dest_hbm, wrep_hbm, xg_hbm, ws_hbm,
               rows_a, rows_b, idx_a, idx_b, wv_a, wv_b, lsem, sem, sem2):
    wid = lax.axis_index("s") * NC + lax.axis_index("c")
    tok0 = (wid % (T // 128)) * 128
    base0 = wid * 128
    nch = 128 // RC
    bufs = [(rows_a, idx_a, wv_a), (rows_b, idx_b, wv_b)]

    def start_load(c, rows_v, idx_v, wv_v):
        base = base0 + c * RC
        tok = tok0 + c * RC
        return (pltpu.async_copy(x_hbm.at[pl.ds(tok, RC)], rows_v, lsem),
                pltpu.async_copy(dest_hbm.at[pl.ds(base, RC)], idx_v, lsem),
                pltpu.async_copy(wrep_hbm.at[pl.ds(base, RC)], wv_v, lsem))

    loads = start_load(0, *bufs[0])
    scat = [None, None]
    for c in range(nch):
        for cp in loads:
            cp.wait()
        rows_v, idx_v, wv_v = bufs[c % 2]
        scat[c % 2] = (pltpu.async_copy(rows_v, xg_hbm.at[idx_v], sem),
                       pltpu.async_copy(wv_v, ws_hbm.at[idx_v], sem2))
        if c + 1 < nch:
            b2 = (c + 1) % 2
            if scat[b2] is not None:
                for cp in scat[b2]:
                    cp.wait()
                scat[b2] = None
            loads = start_load(c + 1, *bufs[b2])
    for pair in scat:
        if pair is not None:
            for cp in pair:
                cp.wait()


def _gmm_body(wt_ref, we_ref, lo_ref, hi_ref,
              xg_ref, w1_ref, b1_ref, w2_ref, b2_ref, ws_ref, out_ref,
              acc_ref):
    d = pl.program_id(0)
    i = pl.program_id(1)
    lo = lo_ref[i]
    hi = hi_ref[i]
    tile = wt_ref[i]

    @pl.when(hi > lo)
    def _():
        row = tile * TT + lax.broadcasted_iota(jnp.int32, (TT, 1), 0)
        mask = (row >= lo) & (row < hi)
        base = pl.multiple_of(tile * TT, TT)
        h = jnp.dot(xg_ref[...], w1_ref[0],
                    preferred_element_type=jnp.float32)
        h = h + b1_ref[0]
        h = h * (1.0 / (1.0 + jnp.exp(-h)))
        contrib = jnp.dot(h, w2_ref[0], preferred_element_type=jnp.float32)
        aslice = acc_ref[pl.ds(base, TT), :]

        @pl.when(d < ND - 1)
        def _():
            acc_ref[pl.ds(base, TT), :] = jnp.where(
                mask, jnp.where(d == 0, contrib, aslice + contrib), aslice)

        @pl.when(d == ND - 1)
        def _():
            wcol = ws_ref[...][:, :1]
            out_ref[...] = jnp.where(
                mask, (aslice + contrib + b2_ref[0]) * wcol, out_ref[...])


RCC = 16


def _comb_body(y_hbm, dest_hbm, out_hbm,
               r0_a, r1_a, r0_b, r1_b, i0_a, i1_a, i0_b, i1_b, g0, g1):
    wid = lax.axis_index("s") * NC + lax.axis_index("c")
    tok0 = wid * (T // NW)
    nch = (T // NW) // RCC
    bufs = [(r0_a, r1_a, i0_a, i1_a), (r0_b, r1_b, i0_b, i1_b)]

    def start(c, r0_v, r1_v, i0_v, i1_v):
        tok = tok0 + c * RCC
        pltpu.sync_copy(dest_hbm.at[pl.ds(tok, RCC)], i0_v)
        pltpu.sync_copy(dest_hbm.at[pl.ds(T + tok, RCC)], i1_v)
        return (pltpu.async_copy(y_hbm.at[i0_v], r0_v, g0),
                pltpu.async_copy(y_hbm.at[i1_v], r1_v, g1))

    cps = start(0, *bufs[0])
    for c in range(nch):
        r0_v, r1_v, _, _ = bufs[c % 2]
        for cp in cps:
            cp.wait()
        if c + 1 < nch:
            cps = start(c + 1, *bufs[(c + 1) % 2])

        def body(r, carry):
            for cc in range(D // 16):
                sl = pl.ds(cc * 16, 16)
                r0_v[r, sl] = r0_v[r, sl] + r1_v[r, sl]
            return carry

        lax.fori_loop(0, RCC, body, 0)
        pltpu.sync_copy(r0_v, out_hbm.at[pl.ds(tok0 + c * RCC, RCC)])


@jax.jit
def kernel(x, Wg, bg, Wng, bng, W1, b1, W2, b2):
    xf = x.reshape(T, D)
    gate = jnp.einsum('bsd,de->bse', x, Wg) + bg
    noise = jax.random.normal(jax.random.key(42), gate.shape, dtype=gate.dtype)
    logits = gate + noise + jnp.einsum('bsd,de->bse', x, Wng) + bng
    lgp = jnp.full((T, EP), _NEG, jnp.float32).at[:, :E].set(logits.reshape(T, E))

    i1, i2, w1_, w2_ = pl.pallas_call(
        _router_body,
        out_shape=[
            jax.ShapeDtypeStruct((T, 1), jnp.int32),
            jax.ShapeDtypeStruct((T, 1), jnp.int32),
            jax.ShapeDtypeStruct((T, 1), jnp.float32),
            jax.ShapeDtypeStruct((T, 1), jnp.float32),
        ],
    )(lgp)

    eidp = jnp.concatenate([i1[:, 0], i2[:, 0]]).reshape(32, 128)
    wflat = jnp.concatenate([w1_[:, 0], w2_[:, 0]])
    wrep = jnp.broadcast_to(wflat[:, None], (TK, 128))

    dest32, wi = pl.pallas_call(
        _rank_body,
        out_shape=[
            jax.ShapeDtypeStruct((32, 128), jnp.int32),
            jax.ShapeDtypeStruct((4, 32), jnp.int32),
        ],
    )(eidp)
    dest = dest32.reshape(TK)
    wi_t, wi_e, wi_lo, wi_hi = wi[0], wi[1], wi[2], wi[3]

    xg, wsrt = pl.kernel(
        _disp_body,
        out_type=[
            jax.ShapeDtypeStruct((TK, D), jnp.float32),
            jax.ShapeDtypeStruct((TK, 128), jnp.float32),
        ],
        mesh=plsc.VectorSubcoreMesh(core_axis_name="c", subcore_axis_name="s", num_cores=NC, num_subcores=NS),
        scratch_types=[
            pltpu.VMEM((RC, D), jnp.float32),
            pltpu.VMEM((RC, D), jnp.float32),
            pltpu.VMEM((RC,), jnp.int32),
            pltpu.VMEM((RC,), jnp.int32),
            pltpu.VMEM((RC, 128), jnp.float32),
            pltpu.VMEM((RC, 128), jnp.float32),
            pltpu.SemaphoreType.DMA,
            pltpu.SemaphoreType.DMA,
            pltpu.SemaphoreType.DMA,
        ],
    )(xf, dest, wrep)

    ysort = pl.pallas_call(
        _gmm_body,
        grid_spec=pltpu.PrefetchScalarGridSpec(
            num_scalar_prefetch=4,
            grid=(ND, NWI),
            in_specs=[
                pl.BlockSpec((TT, D), lambda d, i, wt, we, lo, hi: (wt[i], 0)),
                pl.BlockSpec((1, D, DC), lambda d, i, wt, we, lo, hi:
                             (we[i], 0, d)),
                pl.BlockSpec((1, 1, DC), lambda d, i, wt, we, lo, hi:
                             (we[i], 0, d)),
                pl.BlockSpec((1, DC, D), lambda d, i, wt, we, lo, hi:
                             (we[i], d, 0)),
                pl.BlockSpec((1, 1, D), lambda d, i, wt, we, lo, hi: (we[i], 0, 0)),
                pl.BlockSpec((TT, 128), lambda d, i, wt, we, lo, hi: (wt[i], 0)),
            ],
            out_specs=pl.BlockSpec(
                (TT, D),
                lambda d, i, wt, we, lo, hi: (jnp.where(d == ND - 1, wt[i], 0), 0)),
            scratch_shapes=[pltpu.VMEM((TK, D), jnp.float32)],
        ),
        out_shape=jax.ShapeDtypeStruct((TK, D), jnp.float32),
        compiler_params=pltpu.CompilerParams(
            dimension_semantics=("arbitrary", "arbitrary"),
        ),
    )(wi_t, wi_e, wi_lo, wi_hi, xg, W1, b1.reshape(E, 1, DFF), W2,
      b2.reshape(E, 1, D), wsrt)

    out = pl.kernel(
        _comb_body,
        out_type=jax.ShapeDtypeStruct((T, D), jnp.float32),
        mesh=plsc.VectorSubcoreMesh(core_axis_name="c", subcore_axis_name="s", num_cores=NC, num_subcores=NS),
        scratch_types=[
            pltpu.VMEM((RCC, D), jnp.float32),
            pltpu.VMEM((RCC, D), jnp.float32),
            pltpu.VMEM((RCC, D), jnp.float32),
            pltpu.VMEM((RCC, D), jnp.float32),
            pltpu.VMEM((RCC,), jnp.int32),
            pltpu.VMEM((RCC,), jnp.int32),
            pltpu.VMEM((RCC,), jnp.int32),
            pltpu.VMEM((RCC,), jnp.int32),
            pltpu.SemaphoreType.DMA,
            pltpu.SemaphoreType.DMA,
        ],
    )(ysort, dest)

    return out.reshape(B, S, D)
